# Initial kernel scaffold; baseline (speedup 1.0000x reference)
import jax
import jax.numpy as jnp
from jax.experimental import pallas as pl


def kernel(x, Wg, bg, We, be):
    raise NotImplementedError("write your pallas kernel here")


# speedup vs baseline: 1.7957x; 1.7957x over previous
import functools

import jax
import jax.numpy as jnp
from jax.experimental import pallas as pl
from jax.experimental.pallas import tpu as pltpu

_E = 8
_NEG_INF = -1e30


def _moe_body(x_ref, Wg_ref, bg_ref, We_ref, be_ref, out_ref, aux_ref,
              w_scr, probs_acc, cnt_acc, *, blk, n_tokens):
    t = pl.program_id(0)
    e = pl.program_id(1)
    nt = pl.num_programs(0)

    @pl.when(e == 0)
    def _gate():
        xb = x_ref[...]
        logits = jax.lax.dot_general(
            xb, Wg_ref[...], (((1,), (0,)), ((), ())),
            preferred_element_type=jnp.float32) + bg_ref[...]
        iota_e = jax.lax.broadcasted_iota(jnp.int32, (blk, _E), 1)
        max1 = jnp.max(logits, axis=1, keepdims=True)
        idx1 = jnp.min(jnp.where(logits == max1, iota_e, _E), axis=1,
                       keepdims=True)
        masked = jnp.where(iota_e == idx1, _NEG_INF, logits)
        max2 = jnp.max(masked, axis=1, keepdims=True)
        idx2 = jnp.min(jnp.where(masked == max2, iota_e, _E), axis=1,
                       keepdims=True)
        e2 = jnp.exp(max2 - max1)
        w1 = 1.0 / (1.0 + e2)
        w2 = 1.0 - w1
        w_scr[...] = (jnp.where(iota_e == idx1, w1, 0.0) +
                      jnp.where(iota_e == idx2, w2, 0.0))
        probs = jnp.exp(logits - max1)
        probs = probs / jnp.sum(probs, axis=1, keepdims=True)
        block_probs = jnp.sum(probs, axis=0, keepdims=True)
        block_cnt = jnp.sum((iota_e == idx1).astype(jnp.float32), axis=0,
                            keepdims=True)

        @pl.when(t == 0)
        def _init():
            probs_acc[...] = block_probs
            cnt_acc[...] = block_cnt

        @pl.when(t > 0)
        def _accum():
            probs_acc[...] += block_probs
            cnt_acc[...] += block_cnt

    acc = jax.lax.dot_general(
        x_ref[...], We_ref[0], (((1,), (0,)), ((), ())),
        preferred_element_type=jnp.float32)
    onehot = (jax.lax.broadcasted_iota(jnp.int32, (_E, 1), 0) == e
              ).astype(jnp.float32)
    w_col = jax.lax.dot_general(w_scr[...], onehot, (((1,), (0,)), ((), ())),
                                preferred_element_type=jnp.float32)
    contrib = (acc + be_ref[0]) * w_col

    @pl.when(e == 0)
    def _first():
        out_ref[...] = contrib

    @pl.when(e > 0)
    def _rest():
        out_ref[...] += contrib

    @pl.when((t == nt - 1) & (e == _E - 1))
    def _aux():
        tokens_per_expert = cnt_acc[...]
        avg_prob = probs_acc[...] / n_tokens
        aux_ref[...] = jnp.sum(
            tokens_per_expert / (n_tokens + 1e-8) * avg_prob,
            axis=1, keepdims=True) * _E


def kernel(x, Wg, bg, We, be):
    n, d = x.shape
    blk = 2048
    nt = n // blk
    grid = (nt, _E)
    body = functools.partial(_moe_body, blk=blk, n_tokens=n)
    out, aux = pl.pallas_call(
        body,
        grid=grid,
        in_specs=[
            pl.BlockSpec((blk, d), lambda t, e: (t, 0)),
            pl.BlockSpec((d, _E), lambda t, e: (0, 0)),
            pl.BlockSpec((1, _E), lambda t, e: (0, 0)),
            pl.BlockSpec((1, d, d), lambda t, e: (e, 0, 0)),
            pl.BlockSpec((1, 1, d), lambda t, e: (e, 0, 0)),
        ],
        out_specs=[
            pl.BlockSpec((blk, d), lambda t, e: (t, 0)),
            pl.BlockSpec((1, 1), lambda t, e: (0, 0)),
        ],
        out_shape=[
            jax.ShapeDtypeStruct((n, d), jnp.float32),
            jax.ShapeDtypeStruct((1, 1), jnp.float32),
        ],
        scratch_shapes=[
            pltpu.VMEM((blk, _E), jnp.float32),
            pltpu.VMEM((1, _E), jnp.float32),
            pltpu.VMEM((1, _E), jnp.float32),
        ],
        compiler_params=pltpu.CompilerParams(
            dimension_semantics=("arbitrary", "arbitrary")),
    )(x, Wg, bg.reshape(1, _E), We, be.reshape(_E, 1, d))
    return out, aux[0, 0]
